# baseline (speedup 1.0000x reference)
import functools

import jax
import jax.numpy as jnp
from jax.experimental import pallas as pl
from jax.experimental.pallas import tpu as pltpu

N_USERS = 5000
N_ITEMS = 5000
N = N_USERS + N_ITEMS
D = 16
EPS = 1e-12

BM = 200
NM = N // BM


def _layer1_body(a_ref, cur_ref, out_ref):
    acc = jnp.dot(a_ref[...], cur_ref[...], preferred_element_type=jnp.float32)
    nrm = jnp.sqrt(jnp.sum(acc * acc, axis=1, keepdims=True))
    out_ref[...] = acc / jnp.maximum(nrm, EPS)


def _layer2_body(a_ref, cur_ref, e0_ref, c1_ref, out_ref):
    acc = jnp.dot(a_ref[...], cur_ref[...], preferred_element_type=jnp.float32)
    nrm = jnp.sqrt(jnp.sum(acc * acc, axis=1, keepdims=True))
    c2 = acc / jnp.maximum(nrm, EPS)
    out_ref[...] = (e0_ref[...] + c1_ref[...] + c2) * (1.0 / 3.0)


def _propagate1(a, e0, interpret=False):
    return pl.pallas_call(
        _layer1_body,
        grid=(NM,),
        in_specs=[
            pl.BlockSpec((BM, N), lambda m: (m, 0)),
            pl.BlockSpec((N, D), lambda m: (0, 0)),
        ],
        out_specs=pl.BlockSpec((BM, D), lambda m: (m, 0)),
        out_shape=jax.ShapeDtypeStruct((N, D), jnp.float32),
        interpret=interpret,
    )(a, e0)


def _propagate2(a, c1, e0, interpret=False):
    return pl.pallas_call(
        _layer2_body,
        grid=(NM,),
        in_specs=[
            pl.BlockSpec((BM, N), lambda m: (m, 0)),
            pl.BlockSpec((N, D), lambda m: (0, 0)),
            pl.BlockSpec((BM, D), lambda m: (m, 0)),
            pl.BlockSpec((BM, D), lambda m: (m, 0)),
        ],
        out_specs=pl.BlockSpec((BM, D), lambda m: (m, 0)),
        out_shape=jax.ShapeDtypeStruct((N, D), jnp.float32),
        interpret=interpret,
    )(a, c1, e0, c1)


def kernel(user_emb, item_emb, edge_index, user_id, pos_item, neg_item):
    e0 = jnp.concatenate([user_emb, item_emb], axis=0)
    c1 = _propagate1(edge_index, e0)
    all_emb = _propagate2(edge_index, c1, e0)

    u = jnp.take(all_emb, user_id, axis=0)
    p = jnp.take(all_emb, pos_item + N_USERS, axis=0)
    n = jnp.take(all_emb, neg_item + N_USERS, axis=0)
    diff = jnp.sum(u * p, axis=-1) - jnp.sum(u * n, axis=-1)
    rec_loss = -jnp.mean(jnp.log(jax.nn.sigmoid(diff)))
    return (rec_loss, all_emb)


# speedup vs baseline: 1.0461x; 1.0461x over previous
import functools

import jax
import jax.numpy as jnp
from jax.experimental import pallas as pl
from jax.experimental.pallas import tpu as pltpu

N_USERS = 5000
N_ITEMS = 5000
N = N_USERS + N_ITEMS
D = 16
EPS = 1e-12

BM = 400
NM = N // BM


def _gcn_body(a_ref, e0_ref, out_ref, c1_ref):
    l = pl.program_id(0)
    m = pl.program_id(1)
    row = pl.ds(m * BM, BM)
    cur = jnp.where(l == 0, e0_ref[...], c1_ref[...])
    acc = jnp.dot(a_ref[...], cur, preferred_element_type=jnp.float32)
    nrm = jnp.sqrt(jnp.sum(acc * acc, axis=1, keepdims=True))
    c = acc / jnp.maximum(nrm, EPS)

    @pl.when(l == 0)
    def _():
        c1_ref[row, :] = c
        out_ref[...] = c

    @pl.when(l == 1)
    def _():
        out_ref[...] = (e0_ref[row, :] + c1_ref[row, :] + c) * (1.0 / 3.0)


def _propagate(a, e0, interpret=False):
    return pl.pallas_call(
        _gcn_body,
        grid=(2, NM),
        in_specs=[
            pl.BlockSpec((BM, N), lambda l, m: (m, 0)),
            pl.BlockSpec((N, D), lambda l, m: (0, 0)),
        ],
        out_specs=pl.BlockSpec((BM, D), lambda l, m: (m, 0)),
        out_shape=jax.ShapeDtypeStruct((N, D), jnp.float32),
        scratch_shapes=[pltpu.VMEM((N, D), jnp.float32)],
        interpret=interpret,
    )(a, e0)


def kernel(user_emb, item_emb, edge_index, user_id, pos_item, neg_item):
    e0 = jnp.concatenate([user_emb, item_emb], axis=0)
    all_emb = _propagate(edge_index, e0)

    u = jnp.take(all_emb, user_id, axis=0)
    p = jnp.take(all_emb, pos_item + N_USERS, axis=0)
    n = jnp.take(all_emb, neg_item + N_USERS, axis=0)
    diff = jnp.sum(u * p, axis=-1) - jnp.sum(u * n, axis=-1)
    rec_loss = -jnp.mean(jnp.log(jax.nn.sigmoid(diff)))
    return (rec_loss, all_emb)


# speedup vs baseline: 1.0900x; 1.0419x over previous
import functools

import jax
import jax.numpy as jnp
from jax import lax
from jax.experimental import pallas as pl
from jax.experimental.pallas import tpu as pltpu
from jax.experimental.pallas import tpu_sc as plsc

N_USERS = 5000
N_ITEMS = 5000
N = N_USERS + N_ITEMS
D = 16
EPS = 1e-12

BM = 400
NM = N // BM


def _gcn_body(a_ref, e0_ref, out_ref, pad_ref, c1_ref):
    l = pl.program_id(0)
    m = pl.program_id(1)
    row = pl.ds(m * BM, BM)
    cur = jnp.where(l == 0, e0_ref[...], c1_ref[...])
    acc = jnp.dot(a_ref[...], cur, preferred_element_type=jnp.float32)
    nrm = jnp.sqrt(jnp.sum(acc * acc, axis=1, keepdims=True))
    c = acc / jnp.maximum(nrm, EPS)

    @pl.when(l == 0)
    def _():
        c1_ref[row, :] = c
        out_ref[...] = c

    @pl.when(l == 1)
    def _():
        mean = (e0_ref[row, :] + c1_ref[row, :] + c) * (1.0 / 3.0)
        out_ref[...] = mean
        pad_ref[:, 0:D] = mean


def _propagate(a, e0, interpret=False):
    return pl.pallas_call(
        _gcn_body,
        grid=(2, NM),
        in_specs=[
            pl.BlockSpec((BM, N), lambda l, m: (m, 0)),
            pl.BlockSpec((N, D), lambda l, m: (0, 0)),
        ],
        out_specs=[
            pl.BlockSpec((BM, D), lambda l, m: (m, 0)),
            pl.BlockSpec((BM, 128), lambda l, m: (m, 0)),
        ],
        out_shape=[
            jax.ShapeDtypeStruct((N, D), jnp.float32),
            jax.ShapeDtypeStruct((N, 128), jnp.float32),
        ],
        scratch_shapes=[pltpu.VMEM((N, D), jnp.float32)],
        interpret=interpret,
    )(a, e0)


B = 4096
NW = 32
BPW = B // NW

_SC_MESH = plsc.VectorSubcoreMesh(core_axis_name="c", subcore_axis_name="s")


@functools.partial(
    pl.kernel,
    mesh=_SC_MESH,
    out_type=jax.ShapeDtypeStruct((3 * B, 128), jnp.float32),
    scratch_types=[
        pltpu.VMEM((BPW,), jnp.int32),
        pltpu.VMEM((BPW,), jnp.int32),
        pltpu.VMEM((BPW,), jnp.int32),
        pltpu.VMEM((BPW, 128), jnp.float32),
        pltpu.VMEM((BPW, 128), jnp.float32),
        pltpu.VMEM((BPW, 128), jnp.float32),
        pltpu.SemaphoreType.DMA,
    ],
)
def _sc_gather(emb_hbm, uid_hbm, pid_hbm, nid_hbm, out_hbm,
               uidx, pidx, nidx, urows, prows, nrows, sem):
    wid = lax.axis_index("s") * 2 + lax.axis_index("c")
    base = wid * BPW
    pltpu.sync_copy(uid_hbm.at[pl.ds(base, BPW)], uidx)
    pltpu.sync_copy(pid_hbm.at[pl.ds(base, BPW)], pidx)
    pltpu.sync_copy(nid_hbm.at[pl.ds(base, BPW)], nidx)
    cu = pltpu.async_copy(emb_hbm.at[uidx], urows, sem)
    cp = pltpu.async_copy(emb_hbm.at[pidx], prows, sem)
    cn = pltpu.async_copy(emb_hbm.at[nidx], nrows, sem)
    cu.wait()
    cp.wait()
    cn.wait()
    pltpu.sync_copy(urows, out_hbm.at[pl.ds(base, BPW)])
    pltpu.sync_copy(prows, out_hbm.at[pl.ds(B + base, BPW)])
    pltpu.sync_copy(nrows, out_hbm.at[pl.ds(2 * B + base, BPW)])


def _loss_body(rows_ref, out_ref):
    u = rows_ref[0:B, 0:D]
    p = rows_ref[B:2 * B, 0:D]
    n = rows_ref[2 * B:3 * B, 0:D]
    diff = jnp.sum(u * p, axis=1) - jnp.sum(u * n, axis=1)
    out_ref[0, 0] = -jnp.mean(jnp.log(jax.nn.sigmoid(diff)))


def _loss(rows):
    out = pl.pallas_call(
        _loss_body,
        in_specs=[pl.BlockSpec((3 * B, 128), lambda: (0, 0))],
        out_specs=pl.BlockSpec(memory_space=pltpu.SMEM),
        out_shape=jax.ShapeDtypeStruct((1, 1), jnp.float32),
    )(rows)
    return out[0, 0]


def kernel(user_emb, item_emb, edge_index, user_id, pos_item, neg_item):
    e0 = jnp.concatenate([user_emb, item_emb], axis=0)
    all_emb, pad = _propagate(edge_index, e0)
    rows = _sc_gather(pad, user_id, pos_item + N_USERS, neg_item + N_USERS)
    rec_loss = _loss(rows)
    return (rec_loss, all_emb)
